# baseline (speedup 1.0000x reference)
import functools

import jax
import jax.numpy as jnp
from jax import lax
from jax.experimental import pallas as pl
from jax.experimental.pallas import tpu as pltpu
from jax.experimental.pallas import tpu_sc as plsc

B = 4096
SEQ = 50
D2 = 128
NPOS = 3
NNEG = 25
NCAND = NPOS + NNEG
NW = 32
ROWS_PER_W = B // NW
R = 8
NCHUNK = ROWS_PER_W // R
NPAIR = NCHUNK // 2
NA = 16
NB = 32


def _sc_body(inp_hbm, pos_hbm, neg_hbm,
             embw_hbm, embf_hbm, embr_hbm, out_hbm,
             inp_all, pos_all, neg_all, idxf_all, idxa_all, idxb_all,
             f0, a0, b0, erp0, ern0, pred0,
             f1, a1, b1, erp1, ern1, pred1,
             sem_a, sem_b, sem_p0, sem_p1):
    wid = lax.axis_index("s") * 2 + lax.axis_index("c")
    base = wid * ROWS_PER_W
    lane = lax.iota(jnp.int32, 16)
    last = lane == 15

    bufs0 = (f0, a0, b0, erp0, ern0, pred0)
    bufs1 = (f1, a1, b1, erp1, ern1, pred1)

    pltpu.sync_copy(inp_hbm.at[pl.ds(base * SEQ, ROWS_PER_W * SEQ)], inp_all)
    pltpu.sync_copy(pos_hbm.at[pl.ds(base * NPOS, ROWS_PER_W * NPOS)],
                    pos_all)
    pltpu.sync_copy(neg_hbm.at[pl.ds(base * NNEG, ROWS_PER_W * NNEG)],
                    neg_all)

    fpat = (lane & 7) * SEQ
    apat = 3 * lane + 1
    k = lane & 3
    bpat0 = 6 * (lane >> 2) + 2 + k + (k >> 1)
    j = lane + 16
    k = j & 3
    bpat1 = 6 * (j >> 2) + 2 + k + (k >> 1)
    def buildc(c, carry):
        idxf_all[pl.ds(c * 16, 16)] = plsc.load_gather(
            inp_all, [c * R * SEQ + fpat])
        for r in range(R):
            row = c * R + r
            idxa_all[pl.ds(row * NA, 16)] = plsc.load_gather(
                inp_all, [row * SEQ + apat])
            idxb_all[pl.ds(row * NB, 16)] = plsc.load_gather(
                inp_all, [row * SEQ + bpat0])
            idxb_all[pl.ds(row * NB + 16, 16)] = plsc.load_gather(
                inp_all, [row * SEQ + bpat1])
        return carry

    lax.fori_loop(0, NCHUNK, buildc, 0)

    def fire(c, bufs, sem):
        f_v, a_v, b_v, erp_v, ern_v, _ = bufs
        pltpu.async_copy(embf_hbm.at[idxf_all.at[pl.ds(c * 16, 8)]],
                         f_v, sem)
        pltpu.async_copy(embw_hbm.at[idxa_all.at[pl.ds(c * 128, 128)]],
                         a_v, sem)
        pltpu.async_copy(embw_hbm.at[idxb_all.at[pl.ds(c * 256, 128)]],
                         b_v.at[pl.ds(0, 128)], sem)
        pltpu.async_copy(embw_hbm.at[idxb_all.at[pl.ds(c * 256 + 128, 128)]],
                         b_v.at[pl.ds(128, 128)], sem)
        pltpu.async_copy(embr_hbm.at[pos_all.at[pl.ds(c * 24, 24)]],
                         erp_v, sem)
        pltpu.async_copy(embr_hbm.at[neg_all.at[pl.ds(c * 200, 104)]],
                         ern_v.at[pl.ds(0, 104)], sem)
        pltpu.async_copy(embr_hbm.at[neg_all.at[pl.ds(c * 200 + 104, 96)]],
                         ern_v.at[pl.ds(104, 96)], sem)

    def wait_g(c, bufs, sem):
        f_v, a_v, b_v, erp_v, ern_v, _ = bufs
        pltpu.make_async_copy(embf_hbm.at[idxf_all.at[pl.ds(c * 16, 8)]],
                              f_v, sem).wait()
        pltpu.make_async_copy(embw_hbm.at[idxa_all.at[pl.ds(c * 128, 128)]],
                              a_v, sem).wait()
        pltpu.make_async_copy(embw_hbm.at[idxb_all.at[pl.ds(c * 256, 128)]],
                              b_v.at[pl.ds(0, 128)], sem).wait()
        pltpu.make_async_copy(
            embw_hbm.at[idxb_all.at[pl.ds(c * 256 + 128, 128)]],
            b_v.at[pl.ds(128, 128)], sem).wait()
        pltpu.make_async_copy(embr_hbm.at[pos_all.at[pl.ds(c * 24, 24)]],
                              erp_v, sem).wait()
        pltpu.make_async_copy(embr_hbm.at[neg_all.at[pl.ds(c * 200, 104)]],
                              ern_v.at[pl.ds(0, 104)], sem).wait()
        pltpu.make_async_copy(
            embr_hbm.at[neg_all.at[pl.ds(c * 200 + 104, 96)]],
            ern_v.at[pl.ds(104, 96)], sem).wait()

    def wait_pred(bufs, sem):
        pltpu.make_async_copy(
            bufs[5], out_hbm.at[pl.ds(0, R * NCAND)], sem).wait()

    def compute(c, bufs, sem):
        f_v, a_v, b_v, erp_v, ern_v, pred_v = bufs
        row0 = base + c * R

        def tree_sum(vs):
            while len(vs) > 1:
                nxt = [vs[i] + vs[i + 1] for i in range(0, len(vs) - 1, 2)]
                if len(vs) % 2:
                    nxt.append(vs[-1])
                vs = nxt
            return vs[0]

        def rowbody(r, carry):
            accA = [tree_sum([a_v[r * NA + j, pl.ds(q * 16, 16)]
                              for j in range(NA)]) for q in range(4)]
            accB = [tree_sum([b_v[r * NB + j, pl.ds(q * 16, 16)]
                              for j in range(NB)]) for q in range(4)]
            vv = []
            for q in range(4):
                vv.append((f_v[r, pl.ds(q * 16, 16)]
                           + accA[q] * 0.125) * (1.0 / 3.0))
            for q in range(4):
                vv.append((f_v[r, pl.ds(64 + q * 16, 16)]
                           + accB[q] * 0.0625) * (1.0 / 3.0))
            for g in range(0, NCAND, 4):
                accs = []
                for n in range(g, min(g + 4, NCAND)):
                    if n < NPOS:
                        er, i = erp_v, r * NPOS + n
                    else:
                        er, i = ern_v, r * NNEG + (n - NPOS)
                    accs.append(tree_sum(
                        [er[i, pl.ds(q * 16, 16)] * vv[q]
                         for q in range(8)]))
                tots = [plsc.cumsum(a) for a in accs]
                for t, tot in enumerate(tots):
                    plsc.store_scatter(
                        pred_v,
                        [jnp.full((16,), r * NCAND + g + t, jnp.int32)],
                        tot, mask=last)
            return carry

        lax.fori_loop(0, R, rowbody, 0)
        pltpu.async_copy(pred_v,
                         out_hbm.at[pl.ds(row0 * NCAND, R * NCAND)], sem)

    fire(0, bufs0, sem_a)
    fire(1, bufs1, sem_b)

    def pairstep(p, carry):
        c0 = 2 * p
        wait_g(c0, bufs0, sem_a)

        @pl.when(p > 0)
        def _():
            wait_pred(bufs0, sem_p0)

        compute(c0, bufs0, sem_p0)

        @pl.when(p < NPAIR - 1)
        def _():
            fire(c0 + 2, bufs0, sem_a)

        wait_g(c0 + 1, bufs1, sem_b)

        @pl.when(p > 0)
        def _():
            wait_pred(bufs1, sem_p1)

        compute(c0 + 1, bufs1, sem_p1)

        @pl.when(p < NPAIR - 1)
        def _():
            fire(c0 + 3, bufs1, sem_b)

        return carry

    lax.fori_loop(0, NPAIR, pairstep, 0)
    wait_pred(bufs0, sem_p0)
    wait_pred(bufs1, sem_p1)


def _sc_pred(inp, pos, neg, emb_w, emb_f_w, emb_r_w):
    mesh = plsc.VectorSubcoreMesh(core_axis_name="c", subcore_axis_name="s")
    buf_set = [
        pltpu.VMEM((R, D2), jnp.float32),
        pltpu.VMEM((R * NA, 64), jnp.float32),
        pltpu.VMEM((R * NB, 64), jnp.float32),
        pltpu.VMEM((R * NPOS, D2), jnp.float32),
        pltpu.VMEM((R * NNEG, D2), jnp.float32),
        pltpu.VMEM((R * NCAND,), jnp.float32),
    ]
    fn = functools.partial(
        pl.kernel,
        out_type=jax.ShapeDtypeStruct((B * NCAND,), jnp.float32),
        scratch_types=[
            pltpu.VMEM((ROWS_PER_W * SEQ,), jnp.int32),
            pltpu.VMEM((ROWS_PER_W * NPOS,), jnp.int32),
            pltpu.VMEM((ROWS_PER_W * NNEG,), jnp.int32),
            pltpu.VMEM((NCHUNK * 16,), jnp.int32),
            pltpu.VMEM((ROWS_PER_W * NA,), jnp.int32),
            pltpu.VMEM((ROWS_PER_W * NB,), jnp.int32),
        ] + buf_set + buf_set + [
            pltpu.SemaphoreType.DMA,
            pltpu.SemaphoreType.DMA,
            pltpu.SemaphoreType.DMA,
            pltpu.SemaphoreType.DMA,
        ],
        mesh=mesh,
        compiler_params=pltpu.CompilerParams(needs_layout_passes=False,
                                             use_tc_tiling_on_sc=False),
    )(_sc_body)
    return fn(inp, pos, neg, emb_w, emb_f_w, emb_r_w)


def _loss_body(pred_ref, out_ref):
    x = pred_ref[...]
    i = lax.broadcasted_iota(jnp.int32, x.shape, 0)
    j = lax.broadcasted_iota(jnp.int32, x.shape, 1)
    col = (i * 128 + j) % NCAND
    label = (col < NPOS).astype(jnp.float32)
    p = jnp.clip(jax.nn.sigmoid(x), 1e-7, 1.0 - 1e-7)
    ll = label * jnp.log(p) + (1.0 - label) * jnp.log(1.0 - p)
    out_ref[0, 0] = -jnp.sum(ll) * (1.0 / (B * NCAND))


def _loss(pred_flat):
    out = pl.pallas_call(
        _loss_body,
        out_shape=jax.ShapeDtypeStruct((1, 1), jnp.float32),
        out_specs=pl.BlockSpec(memory_space=pltpu.SMEM),
    )(pred_flat.reshape(B * NCAND // 128, 128))
    return out[0, 0]


def kernel(inp, pos, neg, emb_w, emb_f_w, emb_r_w):
    inp = inp.astype(jnp.int32).reshape(-1)
    pos = pos.astype(jnp.int32).reshape(-1)
    neg = neg.astype(jnp.int32).reshape(-1)
    pred = _sc_pred(inp, pos, neg, emb_w, emb_f_w, emb_r_w)
    return _loss(pred)


# speedup vs baseline: 1.0165x; 1.0165x over previous
import functools

import jax
import jax.numpy as jnp
from jax import lax
from jax.experimental import pallas as pl
from jax.experimental.pallas import tpu as pltpu
from jax.experimental.pallas import tpu_sc as plsc

B = 4096
SEQ = 50
D2 = 128
NPOS = 3
NNEG = 25
NCAND = NPOS + NNEG
NW = 32
ROWS_PER_W = B // NW
R = 8
NCHUNK = ROWS_PER_W // R
NPAIR = NCHUNK // 2
NA = 16
NB = 32


def _sc_body(inp_hbm, pos_hbm, neg_hbm,
             embw_hbm, embf_hbm, embr_hbm, out_hbm,
             inp_all, pos_all, neg_all, idxf_all, idxa_all, idxb_all,
             f0, a0, b0, erp0, ern0, pred0,
             f1, a1, b1, erp1, ern1, pred1,
             sem_a, sem_b, sem_p0, sem_p1):
    wid = lax.axis_index("s") * 2 + lax.axis_index("c")
    base = wid * ROWS_PER_W
    lane = lax.iota(jnp.int32, 16)
    last = lane == 15

    bufs0 = (f0, a0, b0, erp0, ern0, pred0)
    bufs1 = (f1, a1, b1, erp1, ern1, pred1)

    pltpu.sync_copy(inp_hbm.at[pl.ds(base * SEQ, ROWS_PER_W * SEQ)], inp_all)
    pltpu.sync_copy(pos_hbm.at[pl.ds(base * NPOS, ROWS_PER_W * NPOS)],
                    pos_all)
    pltpu.sync_copy(neg_hbm.at[pl.ds(base * NNEG, ROWS_PER_W * NNEG)],
                    neg_all)

    fpat = (lane & 7) * SEQ
    apat = 3 * lane + 1
    k = lane & 3
    bpat0 = 6 * (lane >> 2) + 2 + k + (k >> 1)
    j = lane + 16
    k = j & 3
    bpat1 = 6 * (j >> 2) + 2 + k + (k >> 1)
    def buildc(c, carry):
        idxf_all[pl.ds(c * 16, 16)] = plsc.load_gather(
            inp_all, [c * R * SEQ + fpat])
        for r in range(R):
            row = c * R + r
            idxa_all[pl.ds(row * NA, 16)] = plsc.load_gather(
                inp_all, [row * SEQ + apat])
            idxb_all[pl.ds(row * NB, 16)] = plsc.load_gather(
                inp_all, [row * SEQ + bpat0])
            idxb_all[pl.ds(row * NB + 16, 16)] = plsc.load_gather(
                inp_all, [row * SEQ + bpat1])
        return carry

    lax.fori_loop(0, NCHUNK, buildc, 0)

    def fire(c, bufs, sem):
        f_v, a_v, b_v, erp_v, ern_v, _ = bufs
        pltpu.async_copy(embf_hbm.at[idxf_all.at[pl.ds(c * 16, 8)]],
                         f_v, sem)
        pltpu.async_copy(embw_hbm.at[idxa_all.at[pl.ds(c * 128, 128)]],
                         a_v, sem)
        pltpu.async_copy(embw_hbm.at[idxb_all.at[pl.ds(c * 256, 128)]],
                         b_v.at[pl.ds(0, 128)], sem)
        pltpu.async_copy(embw_hbm.at[idxb_all.at[pl.ds(c * 256 + 128, 128)]],
                         b_v.at[pl.ds(128, 128)], sem)
        pltpu.async_copy(embr_hbm.at[pos_all.at[pl.ds(c * 24, 24)]],
                         erp_v, sem)
        pltpu.async_copy(embr_hbm.at[neg_all.at[pl.ds(c * 200, 104)]],
                         ern_v.at[pl.ds(0, 104)], sem)
        pltpu.async_copy(embr_hbm.at[neg_all.at[pl.ds(c * 200 + 104, 96)]],
                         ern_v.at[pl.ds(104, 96)], sem)

    def wait_g(c, bufs, sem):
        f_v, a_v, b_v, erp_v, ern_v, _ = bufs
        pltpu.make_async_copy(embf_hbm.at[idxf_all.at[pl.ds(c * 16, 8)]],
                              f_v, sem).wait()
        pltpu.make_async_copy(embw_hbm.at[idxa_all.at[pl.ds(c * 128, 128)]],
                              a_v, sem).wait()
        pltpu.make_async_copy(embw_hbm.at[idxb_all.at[pl.ds(c * 256, 128)]],
                              b_v.at[pl.ds(0, 128)], sem).wait()
        pltpu.make_async_copy(
            embw_hbm.at[idxb_all.at[pl.ds(c * 256 + 128, 128)]],
            b_v.at[pl.ds(128, 128)], sem).wait()
        pltpu.make_async_copy(embr_hbm.at[pos_all.at[pl.ds(c * 24, 24)]],
                              erp_v, sem).wait()
        pltpu.make_async_copy(embr_hbm.at[neg_all.at[pl.ds(c * 200, 104)]],
                              ern_v.at[pl.ds(0, 104)], sem).wait()
        pltpu.make_async_copy(
            embr_hbm.at[neg_all.at[pl.ds(c * 200 + 104, 96)]],
            ern_v.at[pl.ds(104, 96)], sem).wait()

    def wait_pred(bufs, sem):
        pltpu.make_async_copy(
            bufs[5], out_hbm.at[pl.ds(0, R * NCAND)], sem).wait()

    def compute(c, bufs, sem):
        f_v, a_v, b_v, erp_v, ern_v, pred_v = bufs
        row0 = base + c * R

        def pair_acc(ref, base, nrows):
            acc = None
            for jp in range(0, nrows, 2):
                pair = [ref[base + jp, pl.ds(q * 16, 16)]
                        + ref[base + jp + 1, pl.ds(q * 16, 16)]
                        for q in range(4)]
                acc = pair if acc is None else [acc[q] + pair[q]
                                                for q in range(4)]
            return acc

        def rowbody(r, carry):
            accA = pair_acc(a_v, r * NA, NA)
            accB = pair_acc(b_v, r * NB, NB)
            vv = []
            for q in range(4):
                vv.append((f_v[r, pl.ds(q * 16, 16)]
                           + accA[q] * 0.125) * (1.0 / 3.0))
            for q in range(4):
                vv.append((f_v[r, pl.ds(64 + q * 16, 16)]
                           + accB[q] * 0.0625) * (1.0 / 3.0))
            def dot4(pairs):
                accs = []
                for er, i in pairs:
                    t01 = (er[i, pl.ds(0, 16)] * vv[0]
                           + er[i, pl.ds(16, 16)] * vv[1])
                    t23 = (er[i, pl.ds(32, 16)] * vv[2]
                           + er[i, pl.ds(48, 16)] * vv[3])
                    t45 = (er[i, pl.ds(64, 16)] * vv[4]
                           + er[i, pl.ds(80, 16)] * vv[5])
                    t67 = (er[i, pl.ds(96, 16)] * vv[6]
                           + er[i, pl.ds(112, 16)] * vv[7])
                    accs.append((t01 + t23) + (t45 + t67))
                return [plsc.cumsum(a) for a in accs]

            tots = dot4([(erp_v, r * NPOS + n) for n in range(NPOS)]
                        + [(ern_v, r * NNEG)])
            for t, tot in enumerate(tots):
                plsc.store_scatter(
                    pred_v, [jnp.full((16,), r * NCAND + t, jnp.int32)],
                    tot, mask=last)

            def gbody(g, carry2):
                i0 = r * NNEG + 1 + g * 4
                tots = dot4([(ern_v, i0 + t) for t in range(4)])
                for t, tot in enumerate(tots):
                    plsc.store_scatter(
                        pred_v,
                        [jnp.full((16,), r * NCAND + 4 + g * 4 + t,
                                  jnp.int32)],
                        tot, mask=last)
                return carry2

            lax.fori_loop(0, (NNEG - 1) // 4, gbody, 0)
            return carry

        lax.fori_loop(0, R, rowbody, 0)
        pltpu.async_copy(pred_v,
                         out_hbm.at[pl.ds(row0 * NCAND, R * NCAND)], sem)

    fire(0, bufs0, sem_a)
    fire(1, bufs1, sem_b)

    def pairstep(p, carry):
        c0 = 2 * p
        wait_g(c0, bufs0, sem_a)

        @pl.when(p > 0)
        def _():
            wait_pred(bufs0, sem_p0)

        compute(c0, bufs0, sem_p0)

        @pl.when(p < NPAIR - 1)
        def _():
            fire(c0 + 2, bufs0, sem_a)

        wait_g(c0 + 1, bufs1, sem_b)

        @pl.when(p > 0)
        def _():
            wait_pred(bufs1, sem_p1)

        compute(c0 + 1, bufs1, sem_p1)

        @pl.when(p < NPAIR - 1)
        def _():
            fire(c0 + 3, bufs1, sem_b)

        return carry

    lax.fori_loop(0, NPAIR, pairstep, 0)
    wait_pred(bufs0, sem_p0)
    wait_pred(bufs1, sem_p1)


def _sc_pred(inp, pos, neg, emb_w, emb_f_w, emb_r_w):
    mesh = plsc.VectorSubcoreMesh(core_axis_name="c", subcore_axis_name="s")
    buf_set = [
        pltpu.VMEM((R, D2), jnp.float32),
        pltpu.VMEM((R * NA, 64), jnp.float32),
        pltpu.VMEM((R * NB, 64), jnp.float32),
        pltpu.VMEM((R * NPOS, D2), jnp.float32),
        pltpu.VMEM((R * NNEG, D2), jnp.float32),
        pltpu.VMEM((R * NCAND,), jnp.float32),
    ]
    fn = functools.partial(
        pl.kernel,
        out_type=jax.ShapeDtypeStruct((B * NCAND,), jnp.float32),
        scratch_types=[
            pltpu.VMEM((ROWS_PER_W * SEQ,), jnp.int32),
            pltpu.VMEM((ROWS_PER_W * NPOS,), jnp.int32),
            pltpu.VMEM((ROWS_PER_W * NNEG,), jnp.int32),
            pltpu.VMEM((NCHUNK * 16,), jnp.int32),
            pltpu.VMEM((ROWS_PER_W * NA,), jnp.int32),
            pltpu.VMEM((ROWS_PER_W * NB,), jnp.int32),
        ] + buf_set + buf_set + [
            pltpu.SemaphoreType.DMA,
            pltpu.SemaphoreType.DMA,
            pltpu.SemaphoreType.DMA,
            pltpu.SemaphoreType.DMA,
        ],
        mesh=mesh,
        compiler_params=pltpu.CompilerParams(needs_layout_passes=False,
                                             use_tc_tiling_on_sc=False),
    )(_sc_body)
    return fn(inp, pos, neg, emb_w, emb_f_w, emb_r_w)


def _loss_body(pred_ref, out_ref):
    x = pred_ref[...]
    i = lax.broadcasted_iota(jnp.int32, x.shape, 0)
    j = lax.broadcasted_iota(jnp.int32, x.shape, 1)
    col = (i * 128 + j) % NCAND
    label = (col < NPOS).astype(jnp.float32)
    p = jnp.clip(jax.nn.sigmoid(x), 1e-7, 1.0 - 1e-7)
    ll = label * jnp.log(p) + (1.0 - label) * jnp.log(1.0 - p)
    out_ref[0, 0] = -jnp.sum(ll) * (1.0 / (B * NCAND))


def _loss(pred_flat):
    out = pl.pallas_call(
        _loss_body,
        out_shape=jax.ShapeDtypeStruct((1, 1), jnp.float32),
        out_specs=pl.BlockSpec(memory_space=pltpu.SMEM),
    )(pred_flat.reshape(B * NCAND // 128, 128))
    return out[0, 0]


def kernel(inp, pos, neg, emb_w, emb_f_w, emb_r_w):
    inp = inp.astype(jnp.int32).reshape(-1)
    pos = pos.astype(jnp.int32).reshape(-1)
    neg = neg.astype(jnp.int32).reshape(-1)
    pred = _sc_pred(inp, pos, neg, emb_w, emb_f_w, emb_r_w)
    return _loss(pred)
